# initial kernel scaffold (unmeasured)
import jax
import jax.numpy as jnp
from jax import lax
from jax.experimental import pallas as pl
from jax.experimental.pallas import tpu as pltpu


def kernel(
    x,
):
    def body(*refs):
        pass

    out_shape = jax.ShapeDtypeStruct(..., jnp.float32)
    return pl.pallas_call(body, out_shape=out_shape)(...)


# baseline (device time: 34254 ns/iter reference)
import jax
import jax.numpy as jnp
from jax import lax
from jax.experimental import pallas as pl
from jax.experimental.pallas import tpu as pltpu

N_STEPS = 5


def kernel(x):
    m, n = x.shape[-2], x.shape[-1]
    x2 = x.reshape(m, n)

    def body(x_ref, out_ref, acc_ref, recv_ref, send_sems, recv_sems):
        my_x = lax.axis_index("x")
        my_y = lax.axis_index("y")
        my_z = lax.axis_index("z")

        acc_ref[...] = x_ref[...].astype(jnp.bfloat16)

        partners = [
            (1 - my_x, my_y, my_z),
            (my_x, jnp.bitwise_xor(my_y, 1), my_z),
            (my_x, jnp.bitwise_xor(my_y, 2), my_z),
            (my_x, my_y, jnp.bitwise_xor(my_z, 1)),
            (my_x, my_y, jnp.bitwise_xor(my_z, 2)),
        ]
        for k, partner in enumerate(partners):
            rdma = pltpu.make_async_remote_copy(
                src_ref=acc_ref,
                dst_ref=recv_ref.at[k],
                send_sem=send_sems.at[k],
                recv_sem=recv_sems.at[k],
                device_id=partner,
                device_id_type=pl.DeviceIdType.MESH,
            )
            rdma.start()
            rdma.wait()
            acc_ref[...] = acc_ref[...] + recv_ref[k]

        out_ref[...] = acc_ref[...].astype(jnp.float32)

    return pl.pallas_call(
        body,
        out_shape=jax.ShapeDtypeStruct((m, n), jnp.float32),
        in_specs=[pl.BlockSpec(memory_space=pltpu.VMEM)],
        out_specs=pl.BlockSpec(memory_space=pltpu.VMEM),
        scratch_shapes=[
            pltpu.VMEM((m, n), jnp.bfloat16),
            pltpu.VMEM((N_STEPS, m, n), jnp.bfloat16),
            pltpu.SemaphoreType.DMA((N_STEPS,)),
            pltpu.SemaphoreType.DMA((N_STEPS,)),
        ],
    )(x2)


# device time: 27297 ns/iter; 1.2549x vs baseline; 1.2549x over previous
import functools

import jax
import jax.numpy as jnp
from jax import lax
from jax.experimental import pallas as pl
from jax.experimental.pallas import tpu as pltpu

N_STEPS = 5


def kernel(x):
    m, n = x.shape[-2], x.shape[-1]
    x2 = x.reshape(m, n)

    def body(x_ref, out_ref, acc_ref, recv_ref, send_sems, recv_sems):
        my_x = lax.axis_index("x")
        my_y = lax.axis_index("y")
        my_z = lax.axis_index("z")

        acc_ref[...] = x_ref[...].astype(jnp.bfloat16)

        partners = [
            (1 - my_x, my_y, my_z),
            (my_x, jnp.bitwise_xor(my_y, 1), my_z),
            (my_x, jnp.bitwise_xor(my_y, 2), my_z),
            (my_x, my_y, jnp.bitwise_xor(my_z, 1)),
            (my_x, my_y, jnp.bitwise_xor(my_z, 2)),
        ]

        barrier_sem = pltpu.get_barrier_semaphore()
        for partner in partners:
            pl.semaphore_signal(
                barrier_sem, inc=1,
                device_id=partner, device_id_type=pl.DeviceIdType.MESH,
            )
        pl.semaphore_wait(barrier_sem, N_STEPS)

        for k, partner in enumerate(partners):
            rdma = pltpu.make_async_remote_copy(
                src_ref=acc_ref,
                dst_ref=recv_ref.at[k],
                send_sem=send_sems.at[k],
                recv_sem=recv_sems.at[k],
                device_id=partner,
                device_id_type=pl.DeviceIdType.MESH,
            )
            rdma.start()
            rdma.wait()
            acc_ref[...] = acc_ref[...] + recv_ref[k]

        out_ref[...] = acc_ref[...].astype(jnp.float32)

        @functools.partial(
            pl.run_scoped, second_barrier=pltpu.SemaphoreType.REGULAR
        )
        def _(second_barrier):
            for partner in partners:
                pl.semaphore_signal(
                    second_barrier, inc=1,
                    device_id=partner, device_id_type=pl.DeviceIdType.MESH,
                )
            pl.semaphore_wait(second_barrier, N_STEPS)

    return pl.pallas_call(
        body,
        out_shape=jax.ShapeDtypeStruct((m, n), jnp.float32),
        in_specs=[pl.BlockSpec(memory_space=pltpu.VMEM)],
        out_specs=pl.BlockSpec(memory_space=pltpu.VMEM),
        scratch_shapes=[
            pltpu.VMEM((m, n), jnp.bfloat16),
            pltpu.VMEM((N_STEPS, m, n), jnp.bfloat16),
            pltpu.SemaphoreType.DMA((N_STEPS,)),
            pltpu.SemaphoreType.DMA((N_STEPS,)),
        ],
        compiler_params=pltpu.CompilerParams(collective_id=0),
    )(x2)


# device time: 22435 ns/iter; 1.5268x vs baseline; 1.2167x over previous
import functools

import jax
import jax.numpy as jnp
from jax import lax
from jax.experimental import pallas as pl
from jax.experimental.pallas import tpu as pltpu

N_PARTNERS = 7


def kernel(x):
    m, n = x.shape[-2], x.shape[-1]
    h = m // 2
    x2 = x.reshape(m, n)

    def body(
        x_ref,
        out_ref,
        acc_ref,
        recv_x,
        recv_y1,
        recv_z1,
        recv_y2,
        recv_z2,
        sem_x,
        send1,
        send2,
        recv_sems1,
        recv_sems2,
    ):
        my_x = lax.axis_index("x")
        my_y = lax.axis_index("y")
        my_z = lax.axis_index("z")
        me = (my_x, my_y, my_z)

        def y_at(off):
            return (my_x, jnp.bitwise_and(my_y + off, 3), my_z)

        def z_at(off):
            return (my_x, my_y, jnp.bitwise_and(my_z + off, 3))

        partners = [(1 - my_x, my_y, my_z)]
        partners += [y_at(j) for j in (1, 2, 3)]
        partners += [z_at(j) for j in (1, 2, 3)]

        barrier_sem = pltpu.get_barrier_semaphore()
        for p in partners:
            pl.semaphore_signal(
                barrier_sem, inc=1,
                device_id=p, device_id_type=pl.DeviceIdType.MESH,
            )
        pl.semaphore_wait(barrier_sem, N_PARTNERS)

        acc_ref[...] = x_ref[...].astype(jnp.bfloat16)

        rdma_x = pltpu.make_async_remote_copy(
            src_ref=acc_ref,
            dst_ref=recv_x,
            send_sem=sem_x.at[0],
            recv_sem=sem_x.at[1],
            device_id=partners[0],
            device_id_type=pl.DeviceIdType.MESH,
        )
        rdma_x.start()
        rdma_x.wait()
        acc_ref[...] = acc_ref[...] + recv_x[...]

        a = pl.ds(0, h)
        b = pl.ds(h, h)

        def bcast(src, dst_slots, send_sems, at, base):
            rdmas = []
            for j in (1, 2, 3):
                r = pltpu.make_async_remote_copy(
                    src_ref=src,
                    dst_ref=dst_slots.at[4 - j],
                    send_sem=send_sems.at[base + j - 1],
                    recv_sem=dst_slots_recv_sem(dst_slots)[0].at[4 - j],
                    device_id=at(j),
                    device_id_type=pl.DeviceIdType.MESH,
                )
                r.start()
                rdmas.append(r)
            return rdmas

        def dst_slots_recv_sem(slots):
            if slots is recv_y1:
                return (recv_sems1.at[0],)
            if slots is recv_z1:
                return (recv_sems1.at[1],)
            if slots is recv_y2:
                return (recv_sems2.at[0],)
            return (recv_sems2.at[1],)

        def wait_recvs(slots):
            sem = dst_slots_recv_sem(slots)[0]
            for s in (1, 2, 3):
                r = pltpu.make_async_remote_copy(
                    src_ref=slots.at[s],
                    dst_ref=slots.at[s],
                    send_sem=sem_x.at[0],
                    recv_sem=sem.at[s],
                    device_id=me,
                    device_id_type=pl.DeviceIdType.MESH,
                )
                r.wait_recv()

        s1 = bcast(acc_ref.at[a], recv_y1, send1, y_at, 0)
        s1 += bcast(acc_ref.at[b], recv_z1, send1, z_at, 3)
        wait_recvs(recv_y1)
        wait_recvs(recv_z1)
        for r in s1:
            r.wait_send()
        acc_ref[a, :] = (
            acc_ref[a, :] + recv_y1[1] + recv_y1[2] + recv_y1[3]
        )
        acc_ref[b, :] = (
            acc_ref[b, :] + recv_z1[1] + recv_z1[2] + recv_z1[3]
        )

        s2 = bcast(acc_ref.at[a], recv_y2, send2, z_at, 0)
        s2 += bcast(acc_ref.at[b], recv_z2, send2, y_at, 3)
        wait_recvs(recv_y2)
        wait_recvs(recv_z2)
        for r in s2:
            r.wait_send()
        acc_ref[a, :] = (
            acc_ref[a, :] + recv_y2[1] + recv_y2[2] + recv_y2[3]
        )
        acc_ref[b, :] = (
            acc_ref[b, :] + recv_z2[1] + recv_z2[2] + recv_z2[3]
        )

        out_ref[...] = acc_ref[...].astype(jnp.float32)

        @functools.partial(
            pl.run_scoped, second_barrier=pltpu.SemaphoreType.REGULAR
        )
        def _(second_barrier):
            for p in partners:
                pl.semaphore_signal(
                    second_barrier, inc=1,
                    device_id=p, device_id_type=pl.DeviceIdType.MESH,
                )
            pl.semaphore_wait(second_barrier, N_PARTNERS)

    return pl.pallas_call(
        body,
        out_shape=jax.ShapeDtypeStruct((m, n), jnp.float32),
        in_specs=[pl.BlockSpec(memory_space=pltpu.VMEM)],
        out_specs=pl.BlockSpec(memory_space=pltpu.VMEM),
        scratch_shapes=[
            pltpu.VMEM((m, n), jnp.bfloat16),
            pltpu.VMEM((m, n), jnp.bfloat16),
            pltpu.VMEM((4, h, n), jnp.bfloat16),
            pltpu.VMEM((4, h, n), jnp.bfloat16),
            pltpu.VMEM((4, h, n), jnp.bfloat16),
            pltpu.VMEM((4, h, n), jnp.bfloat16),
            pltpu.SemaphoreType.DMA((2,)),
            pltpu.SemaphoreType.DMA((6,)),
            pltpu.SemaphoreType.DMA((6,)),
            pltpu.SemaphoreType.DMA((2, 4)),
            pltpu.SemaphoreType.DMA((2, 4)),
        ],
        compiler_params=pltpu.CompilerParams(collective_id=0),
    )(x2)


# device time: 16881 ns/iter; 2.0291x vs baseline; 1.3290x over previous
import jax
import jax.numpy as jnp
from jax import lax
from jax.experimental import pallas as pl
from jax.experimental.pallas import tpu as pltpu

N_PARTNERS = 7


def kernel(x):
    m, n = x.shape[-2], x.shape[-1]
    h = m // 2
    x2 = x.reshape(m, n)

    def body(
        x_ref,
        out_ref,
        acc_ref,
        recv_x,
        recv_a1,
        recv_b1,
        recv_a2,
        recv_b2,
        sem_x,
        send1,
        send2,
        rsem1,
        rsem2,
    ):
        my_x = lax.axis_index("x")
        my_y = lax.axis_index("y")
        my_z = lax.axis_index("z")
        me = (my_x, my_y, my_z)

        def y_at(off):
            return (my_x, jnp.bitwise_and(my_y + off, 3), my_z)

        def z_at(off):
            return (my_x, my_y, jnp.bitwise_and(my_z + off, 3))

        partners = [(1 - my_x, my_y, my_z)]
        partners += [y_at(j) for j in (1, 2, 3)]
        partners += [z_at(j) for j in (1, 2, 3)]

        barrier_sem = pltpu.get_barrier_semaphore()
        for p in partners:
            pl.semaphore_signal(
                barrier_sem, inc=1,
                device_id=p, device_id_type=pl.DeviceIdType.MESH,
            )
        pl.semaphore_wait(barrier_sem, N_PARTNERS)

        a = pl.ds(0, h)
        b = pl.ds(h, h)

        def bcast(src, dst_slots, send_sems, recv_sems, at):
            rdmas = []
            for j in (1, 2, 3):
                r = pltpu.make_async_remote_copy(
                    src_ref=src,
                    dst_ref=dst_slots.at[4 - j],
                    send_sem=send_sems.at[j - 1],
                    recv_sem=recv_sems.at[4 - j],
                    device_id=at(j),
                    device_id_type=pl.DeviceIdType.MESH,
                )
                r.start()
                rdmas.append(r)
            return rdmas

        def wait_recvs(slots, recv_sems):
            for s in (1, 2, 3):
                r = pltpu.make_async_remote_copy(
                    src_ref=slots.at[s],
                    dst_ref=slots.at[s],
                    send_sem=sem_x.at[0, 0],
                    recv_sem=recv_sems.at[s],
                    device_id=me,
                    device_id_type=pl.DeviceIdType.MESH,
                )
                r.wait_recv()

        acc_ref[a, :] = x_ref[a, :].astype(jnp.bfloat16)
        xch_a = pltpu.make_async_remote_copy(
            src_ref=acc_ref.at[a],
            dst_ref=recv_x.at[a],
            send_sem=sem_x.at[0, 0],
            recv_sem=sem_x.at[0, 1],
            device_id=partners[0],
            device_id_type=pl.DeviceIdType.MESH,
        )
        xch_a.start()
        acc_ref[b, :] = x_ref[b, :].astype(jnp.bfloat16)
        xch_b = pltpu.make_async_remote_copy(
            src_ref=acc_ref.at[b],
            dst_ref=recv_x.at[b],
            send_sem=sem_x.at[1, 0],
            recv_sem=sem_x.at[1, 1],
            device_id=partners[0],
            device_id_type=pl.DeviceIdType.MESH,
        )
        xch_b.start()

        xch_a.wait()
        acc_ref[a, :] = acc_ref[a, :] + recv_x[a, :]
        s1a = bcast(acc_ref.at[a], recv_a1, send1.at[0], rsem1.at[0], y_at)

        xch_b.wait()
        acc_ref[b, :] = acc_ref[b, :] + recv_x[b, :]
        s1b = bcast(acc_ref.at[b], recv_b1, send1.at[1], rsem1.at[1], z_at)

        wait_recvs(recv_a1, rsem1.at[0])
        for r in s1a:
            r.wait_send()
        acc_ref[a, :] = acc_ref[a, :] + recv_a1[1] + recv_a1[2] + recv_a1[3]
        s2a = bcast(acc_ref.at[a], recv_a2, send2.at[0], rsem2.at[0], z_at)

        wait_recvs(recv_b1, rsem1.at[1])
        for r in s1b:
            r.wait_send()
        acc_ref[b, :] = acc_ref[b, :] + recv_b1[1] + recv_b1[2] + recv_b1[3]
        s2b = bcast(acc_ref.at[b], recv_b2, send2.at[1], rsem2.at[1], y_at)

        wait_recvs(recv_a2, rsem2.at[0])
        acc_ref[a, :] = acc_ref[a, :] + recv_a2[1] + recv_a2[2] + recv_a2[3]
        out_ref[a, :] = acc_ref[a, :].astype(jnp.float32)

        wait_recvs(recv_b2, rsem2.at[1])
        acc_ref[b, :] = acc_ref[b, :] + recv_b2[1] + recv_b2[2] + recv_b2[3]
        out_ref[b, :] = acc_ref[b, :].astype(jnp.float32)

        for r in s2a + s2b:
            r.wait_send()

    return pl.pallas_call(
        body,
        out_shape=jax.ShapeDtypeStruct((m, n), jnp.float32),
        in_specs=[pl.BlockSpec(memory_space=pltpu.VMEM)],
        out_specs=pl.BlockSpec(memory_space=pltpu.VMEM),
        scratch_shapes=[
            pltpu.VMEM((m, n), jnp.bfloat16),
            pltpu.VMEM((m, n), jnp.bfloat16),
            pltpu.VMEM((4, h, n), jnp.bfloat16),
            pltpu.VMEM((4, h, n), jnp.bfloat16),
            pltpu.VMEM((4, h, n), jnp.bfloat16),
            pltpu.VMEM((4, h, n), jnp.bfloat16),
            pltpu.SemaphoreType.DMA((2, 2)),
            pltpu.SemaphoreType.DMA((2, 3)),
            pltpu.SemaphoreType.DMA((2, 3)),
            pltpu.SemaphoreType.DMA((2, 4)),
            pltpu.SemaphoreType.DMA((2, 4)),
        ],
        compiler_params=pltpu.CompilerParams(collective_id=0),
    )(x2)
